# baseline (speedup 1.0000x reference)
import functools

import jax
import jax.numpy as jnp
from jax import lax
from jax.experimental import pallas as pl
from jax.experimental.pallas import tpu as pltpu
from jax.experimental.pallas import tpu_sc as plsc

BATCH = 16384
FIELDS = 26
DIM = 16
NC = 2
NS = 16
NW = NC * NS
ROWS_PER_W = BATCH // NW
CHUNK = 64
NCHUNK = ROWS_PER_W // CHUNK
GPC = CHUNK // 16
IDX_PER_CHUNK = CHUNK * FIELDS
NDMA = IDX_PER_CHUNK // 128


def _fm_body(x2, emb, lin16, bias16, out, idx_v, idx2_v, rows_v, lin_v,
             out_v, bias_v, sem):
    c = lax.axis_index("c")
    s = lax.axis_index("s")
    wid = s * NC + c

    pltpu.sync_copy(bias16, bias_v)

    iota = lax.broadcasted_iota(jnp.int32, (16,), 0)
    riota26 = iota * FIELDS

    def chunk_body(ci, carry):
        row_base = wid * (NCHUNK * NDMA) + ci * NDMA
        pltpu.sync_copy(x2.at[pl.ds(row_base, NDMA)], idx_v)

        for j in range(NDMA):
            for k in range(8):
                v = idx_v[j, pl.ds(k * 16, 16)]
                idx2_v[j, pl.ds(k * 16, 16)] = lax.shift_right_logical(v, 4)

        cps = []
        for j in range(NDMA):
            cps.append(pltpu.async_copy(
                emb.at[idx_v.at[j]],
                rows_v.at[pl.ds(j * 128, 128)], sem))
            cps.append(pltpu.async_copy(
                lin16.at[idx2_v.at[j]],
                lin_v.at[pl.ds(j * 128, 128)], sem))
        for cp in cps:
            cp.wait()

        def group_body(g, carry2):
            gbase = riota26 + g * (16 * FIELDS)
            s_acc = [jnp.zeros((16,), jnp.float32) for _ in range(DIM)]
            q_acc = [jnp.zeros((16,), jnp.float32) for _ in range(DIM)]
            for f in range(FIELDS):
                rowv = gbase + f
                for d in range(DIM):
                    dvec = jnp.full((16,), d, jnp.int32)
                    v = plsc.load_gather(rows_v, [rowv, dvec])
                    s_acc[d] = s_acc[d] + v
                    q_acc[d] = q_acc[d] + v * v
            ix = jnp.zeros((16,), jnp.float32)
            for d in range(DIM):
                ix = ix + (s_acc[d] * s_acc[d] - q_acc[d])
            lin0 = jnp.zeros((16,), jnp.float32)
            lin1 = jnp.zeros((16,), jnp.float32)
            for f in range(FIELDS):
                p = gbase + f
                orig = plsc.load_gather(idx_v, [lax.shift_right_logical(p, 7),
                                                lax.bitwise_and(p, 127)])
                lo = lax.bitwise_and(orig, 15)
                if f % 2 == 0:
                    lin0 = lin0 + plsc.load_gather(lin_v, [p, lo])
                else:
                    lin1 = lin1 + plsc.load_gather(lin_v, [p, lo])
            z = (lin0 + lin1) + 0.5 * ix + bias_v[...]
            p_out = 1.0 / (1.0 + jnp.exp(-z))
            out_v[pl.ds(g * 16, 16)] = p_out
            return carry2

        lax.fori_loop(0, GPC, group_body, 0)
        pltpu.sync_copy(out_v, out.at[pl.ds(wid * ROWS_PER_W + ci * CHUNK,
                                            CHUNK)])
        return carry

    lax.fori_loop(0, NCHUNK, chunk_body, 0)


@jax.jit
def kernel(x, emb_table, linear_table, bias):
    x2 = x.astype(jnp.int32).reshape(-1, 128)
    lin16 = jnp.take(linear_table[:, 0],
                     jnp.arange(linear_table.shape[0], dtype=jnp.int32)
                     .reshape(-1, 16))
    bias16 = jnp.broadcast_to(bias.astype(jnp.float32), (16,))
    mesh = plsc.VectorSubcoreMesh(core_axis_name="c", subcore_axis_name="s",
                                  num_cores=NC, num_subcores=NS)
    fm = pl.kernel(
        _fm_body,
        out_type=jax.ShapeDtypeStruct((BATCH,), jnp.float32),
        mesh=mesh,
        compiler_params=pltpu.CompilerParams(needs_layout_passes=False,
                                             use_tc_tiling_on_sc=False),
        scratch_types=[
            pltpu.VMEM((NDMA, 128), jnp.int32),
            pltpu.VMEM((NDMA, 128), jnp.int32),
            pltpu.VMEM((IDX_PER_CHUNK, DIM), jnp.float32),
            pltpu.VMEM((IDX_PER_CHUNK, DIM), jnp.float32),
            pltpu.VMEM((CHUNK,), jnp.float32),
            pltpu.VMEM((16,), jnp.float32),
            pltpu.SemaphoreType.DMA,
        ],
    )
    return fm(x2, emb_table, lin16, bias16)


# speedup vs baseline: 1.2914x; 1.2914x over previous
import functools

import jax
import jax.numpy as jnp
from jax import lax
from jax.experimental import pallas as pl
from jax.experimental.pallas import tpu as pltpu
from jax.experimental.pallas import tpu_sc as plsc

BATCH = 16384
FIELDS = 26
DIM = 16
NC = 2
NS = 16
NW = NC * NS
ROWS_PER_W = BATCH // NW
CHUNK = 64
NCHUNK = ROWS_PER_W // CHUNK
GPC = CHUNK // 16
IDX_PER_CHUNK = CHUNK * FIELDS
NDMA = IDX_PER_CHUNK // 128


def _fm_body(x2, emb, lin16, bias16, out, xchunk_v, idx_v, idx2_v, rows_v,
             lin_v, out_v, bias_v, sem):
    c = lax.axis_index("c")
    s = lax.axis_index("s")
    wid = s * NC + c

    pltpu.sync_copy(bias16, bias_v)

    iota = lax.broadcasted_iota(jnp.int32, (16,), 0)
    riota26 = iota * FIELDS

    def chunk_body(ci, carry):
        b0 = wid * ROWS_PER_W + ci * CHUNK
        pltpu.sync_copy(x2.at[pl.ds(b0, CHUNK)], xchunk_v)
        for j in range(NDMA):
            for k in range(8):
                p = iota + (j * 128 + k * 16)
                r = lax.div(p, FIELDS)
                cc = p - r * FIELDS
                v = plsc.load_gather(xchunk_v, [r, cc])
                idx_v[j, pl.ds(k * 16, 16)] = v
                idx2_v[j, pl.ds(k * 16, 16)] = lax.shift_right_logical(v, 4)

        cps = []
        for j in range(NDMA):
            cps.append(pltpu.async_copy(
                emb.at[idx_v.at[j]],
                rows_v.at[pl.ds(j * 128, 128)], sem))
            cps.append(pltpu.async_copy(
                lin16.at[idx2_v.at[j]],
                lin_v.at[pl.ds(j * 128, 128)], sem))
        for cp in cps:
            cp.wait()

        def group_body(g, carry2):
            gbase = riota26 + g * (16 * FIELDS)
            s_acc = [jnp.zeros((16,), jnp.float32) for _ in range(DIM)]
            q_acc = [jnp.zeros((16,), jnp.float32) for _ in range(DIM)]
            for f in range(FIELDS):
                rowv = gbase + f
                for d in range(DIM):
                    dvec = jnp.full((16,), d, jnp.int32)
                    v = plsc.load_gather(rows_v, [rowv, dvec])
                    s_acc[d] = s_acc[d] + v
                    q_acc[d] = q_acc[d] + v * v
            ix = jnp.zeros((16,), jnp.float32)
            for d in range(DIM):
                ix = ix + (s_acc[d] * s_acc[d] - q_acc[d])
            lin0 = jnp.zeros((16,), jnp.float32)
            lin1 = jnp.zeros((16,), jnp.float32)
            for f in range(FIELDS):
                p = gbase + f
                orig = plsc.load_gather(idx_v, [lax.shift_right_logical(p, 7),
                                                lax.bitwise_and(p, 127)])
                lo = lax.bitwise_and(orig, 15)
                if f % 2 == 0:
                    lin0 = lin0 + plsc.load_gather(lin_v, [p, lo])
                else:
                    lin1 = lin1 + plsc.load_gather(lin_v, [p, lo])
            z = (lin0 + lin1) + 0.5 * ix + bias_v[...]
            p_out = 1.0 / (1.0 + jnp.exp(-z))
            out_v[pl.ds(g * 16, 16)] = p_out
            return carry2

        lax.fori_loop(0, GPC, group_body, 0)
        pltpu.sync_copy(out_v, out.at[pl.ds(wid * ROWS_PER_W + ci * CHUNK,
                                            CHUNK)])
        return carry

    lax.fori_loop(0, NCHUNK, chunk_body, 0)


@jax.jit
def kernel(x, emb_table, linear_table, bias):
    x2 = x.astype(jnp.int32)
    lin16 = linear_table.reshape(-1, 16)
    bias16 = jnp.broadcast_to(bias.astype(jnp.float32), (16,))
    mesh = plsc.VectorSubcoreMesh(core_axis_name="c", subcore_axis_name="s",
                                  num_cores=NC, num_subcores=NS)
    fm = pl.kernel(
        _fm_body,
        out_type=jax.ShapeDtypeStruct((BATCH,), jnp.float32),
        mesh=mesh,
        compiler_params=pltpu.CompilerParams(needs_layout_passes=False,
                                             use_tc_tiling_on_sc=False),
        scratch_types=[
            pltpu.VMEM((CHUNK, FIELDS), jnp.int32),
            pltpu.VMEM((NDMA, 128), jnp.int32),
            pltpu.VMEM((NDMA, 128), jnp.int32),
            pltpu.VMEM((IDX_PER_CHUNK, DIM), jnp.float32),
            pltpu.VMEM((IDX_PER_CHUNK, DIM), jnp.float32),
            pltpu.VMEM((CHUNK,), jnp.float32),
            pltpu.VMEM((16,), jnp.float32),
            pltpu.SemaphoreType.DMA,
        ],
    )
    return fm(x2, emb_table, lin16, bias16)


# speedup vs baseline: 1.3567x; 1.0505x over previous
import functools

import jax
import jax.numpy as jnp
from jax import lax
from jax.experimental import pallas as pl
from jax.experimental.pallas import tpu as pltpu
from jax.experimental.pallas import tpu_sc as plsc

BATCH = 16384
FIELDS = 26
DIM = 16
NC = 2
NS = 16
NW = NC * NS
ROWS_PER_W = BATCH // NW
CHUNK = 64
NCHUNK = ROWS_PER_W // CHUNK
NPAIR = NCHUNK // 2
GPC = CHUNK // 16
IDX_PER_CHUNK = CHUNK * FIELDS
NDMA = IDX_PER_CHUNK // 128


def _fm_body(x2, emb, lin16, bias16, out,
             xchunk_v, idxA, idx2A, rowsA, linA, idxB, idx2B, rowsB, linB,
             out_v, bias_v, semA, semB):
    c = lax.axis_index("c")
    s = lax.axis_index("s")
    wid = s * NC + c

    pltpu.sync_copy(bias16, bias_v)

    iota = lax.broadcasted_iota(jnp.int32, (16,), 0)
    riota26 = iota * FIELDS

    def stage_fire(ci, idx_v, idx2_v, rows_v, lin_v, sem):
        b0 = wid * ROWS_PER_W + ci * CHUNK
        pltpu.sync_copy(x2.at[pl.ds(b0, CHUNK)], xchunk_v)
        for j in range(NDMA):
            for k in range(8):
                p = iota + (j * 128 + k * 16)
                r = lax.div(p, FIELDS)
                cc = p - r * FIELDS
                v = plsc.load_gather(xchunk_v, [r, cc])
                idx_v[j, pl.ds(k * 16, 16)] = v
                idx2_v[j, pl.ds(k * 16, 16)] = lax.shift_right_logical(v, 4)
        for j in range(NDMA):
            pltpu.async_copy(emb.at[idx_v.at[j]],
                             rows_v.at[pl.ds(j * 128, 128)], sem)
            pltpu.async_copy(lin16.at[idx2_v.at[j]],
                             lin_v.at[pl.ds(j * 128, 128)], sem)

    def drain(idx_v, idx2_v, rows_v, lin_v, sem):
        for j in range(NDMA):
            pltpu.make_async_copy(emb.at[idx_v.at[j]],
                                  rows_v.at[pl.ds(j * 128, 128)], sem).wait()
            pltpu.make_async_copy(lin16.at[idx2_v.at[j]],
                                  lin_v.at[pl.ds(j * 128, 128)], sem).wait()

    def compute(ci, idx_v, rows_v, lin_v):
        def group_body(g, carry):
            gbase = riota26 + g * (16 * FIELDS)
            s_acc = [jnp.zeros((16,), jnp.float32) for _ in range(DIM)]
            q_acc = [jnp.zeros((16,), jnp.float32) for _ in range(DIM)]
            for f in range(FIELDS):
                rowv = gbase + f
                for d in range(DIM):
                    dvec = jnp.full((16,), d, jnp.int32)
                    v = plsc.load_gather(rows_v, [rowv, dvec])
                    s_acc[d] = s_acc[d] + v
                    q_acc[d] = q_acc[d] + v * v
            ix = jnp.zeros((16,), jnp.float32)
            for d in range(DIM):
                ix = ix + (s_acc[d] * s_acc[d] - q_acc[d])
            lin0 = jnp.zeros((16,), jnp.float32)
            lin1 = jnp.zeros((16,), jnp.float32)
            for f in range(FIELDS):
                p = gbase + f
                orig = plsc.load_gather(
                    idx_v, [lax.shift_right_logical(p, 7),
                            lax.bitwise_and(p, 127)])
                lo = lax.bitwise_and(orig, 15)
                if f % 2 == 0:
                    lin0 = lin0 + plsc.load_gather(lin_v, [p, lo])
                else:
                    lin1 = lin1 + plsc.load_gather(lin_v, [p, lo])
            z = (lin0 + lin1) + 0.5 * ix + bias_v[...]
            p_out = 1.0 / (1.0 + jnp.exp(-z))
            out_v[pl.ds(g * 16, 16)] = p_out
            return carry

        lax.fori_loop(0, GPC, group_body, 0)
        pltpu.sync_copy(out_v, out.at[pl.ds(wid * ROWS_PER_W + ci * CHUNK,
                                            CHUNK)])

    stage_fire(0, idxA, idx2A, rowsA, linA, semA)

    def pair_body(pp, carry):
        ca = 2 * pp
        stage_fire(ca + 1, idxB, idx2B, rowsB, linB, semB)
        drain(idxA, idx2A, rowsA, linA, semA)
        compute(ca, idxA, rowsA, linA)

        @pl.when(pp < NPAIR - 1)
        def _():
            stage_fire(ca + 2, idxA, idx2A, rowsA, linA, semA)

        drain(idxB, idx2B, rowsB, linB, semB)
        compute(ca + 1, idxB, rowsB, linB)
        return carry

    lax.fori_loop(0, NPAIR, pair_body, 0)


@jax.jit
def kernel(x, emb_table, linear_table, bias):
    x2 = x.astype(jnp.int32)
    lin16 = linear_table.reshape(-1, 16)
    bias16 = jnp.broadcast_to(bias.astype(jnp.float32), (16,))
    mesh = plsc.VectorSubcoreMesh(core_axis_name="c", subcore_axis_name="s",
                                  num_cores=NC, num_subcores=NS)
    fm = pl.kernel(
        _fm_body,
        out_type=jax.ShapeDtypeStruct((BATCH,), jnp.float32),
        mesh=mesh,
        compiler_params=pltpu.CompilerParams(needs_layout_passes=False,
                                             use_tc_tiling_on_sc=False),
        scratch_types=[
            pltpu.VMEM((CHUNK, FIELDS), jnp.int32),
            pltpu.VMEM((NDMA, 128), jnp.int32),
            pltpu.VMEM((NDMA, 128), jnp.int32),
            pltpu.VMEM((IDX_PER_CHUNK, DIM), jnp.float32),
            pltpu.VMEM((IDX_PER_CHUNK, DIM), jnp.float32),
            pltpu.VMEM((NDMA, 128), jnp.int32),
            pltpu.VMEM((NDMA, 128), jnp.int32),
            pltpu.VMEM((IDX_PER_CHUNK, DIM), jnp.float32),
            pltpu.VMEM((IDX_PER_CHUNK, DIM), jnp.float32),
            pltpu.VMEM((CHUNK,), jnp.float32),
            pltpu.VMEM((16,), jnp.float32),
            pltpu.SemaphoreType.DMA,
            pltpu.SemaphoreType.DMA,
        ],
    )
    return fm(x2, emb_table, lin16, bias16)
